# baseline (speedup 1.0000x reference)
import functools

import jax
import jax.numpy as jnp
from jax import lax
from jax.experimental import pallas as pl
from jax.experimental.pallas import tpu as pltpu
from jax.experimental.pallas import tpu_sc as plsc

_D = 16
_N_COL = 32
_B = 16384
_NC, _NS = 2, 16
_NW = _NC * _NS
_BPW = _B // _NW
_CHUNK = 128
_NCHUNK = _BPW // _CHUNK


def _sc_lookup(idx3, table):
    mesh = plsc.VectorSubcoreMesh(core_axis_name="c", subcore_axis_name="s")

    @functools.partial(
        pl.kernel,
        mesh=mesh,
        out_type=jax.ShapeDtypeStruct((_B, _N_COL), jnp.float32),
        compiler_params=pltpu.CompilerParams(use_tc_tiling_on_sc=False),
        scratch_types=[
            pltpu.VMEM((_NCHUNK, _CHUNK), jnp.int32),
            pltpu.VMEM((_BPW, _D), jnp.float32),
            pltpu.VMEM((_BPW, _N_COL), jnp.float32),
            pltpu.SemaphoreType.DMA,
        ],
    )
    def body(idx_hbm, table_hbm, out_hbm, idx_v, rows_v, big_v, sem):
        wid = lax.axis_index("s") * _NC + lax.axis_index("c")
        base = wid * _BPW

        pltpu.sync_copy(idx_hbm.at[wid], idx_v)

        copies = [
            pltpu.async_copy(
                table_hbm.at[idx_v.at[j]],
                rows_v.at[pl.ds(j * _CHUNK, _CHUNK)],
                sem,
            )
            for j in range(_NCHUNK)
        ]

        zeros = jnp.zeros((_D,), jnp.float32)

        def zero_body(i, _):
            big_v[i, pl.ds(_D, _D)] = zeros
            return _

        lax.fori_loop(0, _BPW, zero_body, None)

        for c in copies:
            c.wait()

        def asm_body(i, _):
            big_v[i, pl.ds(0, _D)] = rows_v[i, :]
            return _

        lax.fori_loop(0, _BPW, asm_body, None)

        pltpu.sync_copy(big_v, out_hbm.at[pl.ds(base, _BPW)])

    return body(idx3, table)


def kernel(index, table):
    idx3 = index.astype(jnp.int32).reshape(_NW, _NCHUNK, _CHUNK)
    return _sc_lookup(idx3, table)


# speedup vs baseline: 4.6224x; 4.6224x over previous
import functools

import jax
import jax.numpy as jnp
from jax import lax
from jax.experimental import pallas as pl
from jax.experimental.pallas import tpu as pltpu
from jax.experimental.pallas import tpu_sc as plsc

_VOCAB = 1000000
_D = 16
_N_COL = 32
_B = 16384
_NC, _NS = 2, 16
_NW = _NC * _NS
_BPW = _B // _NW
_G = 16


def _sc_lookup(index, table3):
    mesh = plsc.VectorSubcoreMesh(core_axis_name="c", subcore_axis_name="s")

    @functools.partial(
        pl.kernel,
        mesh=mesh,
        out_type=jax.ShapeDtypeStruct((_B, _N_COL), jnp.float32),
        compiler_params=pltpu.CompilerParams(
            needs_layout_passes=False, use_tc_tiling_on_sc=True
        ),
        scratch_types=[
            pltpu.VMEM((_BPW,), jnp.int32),
            pltpu.VMEM((_G, 2, 8, 128), jnp.float32),
            pltpu.VMEM((_BPW, _N_COL), jnp.float32),
            pltpu.SemaphoreType.DMA,
        ],
    )
    def body(idx_hbm, table_hbm, out_hbm, idx_v, tiles_v, big_v, sem):
        wid = lax.axis_index("s") * _NC + lax.axis_index("c")
        base = wid * _BPW

        pltpu.sync_copy(idx_hbm.at[pl.ds(base, _BPW)], idx_v)

        lanes = lax.iota(jnp.int32, 16)
        plane = lanes >> 3
        sub = lanes & 7
        zeros = jnp.zeros((_D,), jnp.float32)

        def grp_body(g, _):
            i0 = g * _G
            idx16 = idx_v[pl.ds(i0, _G)]
            for k in range(_G):
                r = idx16[k]
                r_al = pl.multiple_of((r >> 7) << 7, 128)
                pltpu.async_copy(
                    table_hbm.at[0, :, pl.ds(r_al, 128)], tiles_v.at[k, 0], sem
                )
                pltpu.async_copy(
                    table_hbm.at[1, :, pl.ds(r_al, 128)], tiles_v.at[k, 1], sem
                )
            for k in range(_G):
                pltpu.make_async_copy(
                    table_hbm.at[0, :, pl.ds(0, 128)], tiles_v.at[k, 0], sem
                ).wait()
                pltpu.make_async_copy(
                    table_hbm.at[1, :, pl.ds(0, 128)], tiles_v.at[k, 1], sem
                ).wait()
            off16 = idx16 & 127
            for k in range(_G):
                i = i0 + k
                roff = jnp.full((16,), off16[k], jnp.int32)
                vals = plsc.load_gather(
                    tiles_v, [jnp.full((16,), k, jnp.int32), plane, sub, roff]
                )
                big_v[i, pl.ds(0, _D)] = vals
                big_v[i, pl.ds(_D, _D)] = zeros
            return _

        lax.fori_loop(0, _BPW // _G, grp_body, None)

        pltpu.sync_copy(big_v, out_hbm.at[pl.ds(base, _BPW)])

    return body(index, table3)


def kernel(index, table):
    table3 = table.T.reshape(2, 8, _VOCAB)
    return _sc_lookup(index.astype(jnp.int32), table3)


# speedup vs baseline: 5.6110x; 1.2139x over previous
import functools

import jax
import jax.numpy as jnp
from jax import lax
from jax.experimental import pallas as pl
from jax.experimental.pallas import tpu as pltpu
from jax.experimental.pallas import tpu_sc as plsc

_VOCAB = 1000000
_D = 16
_N_COL = 32
_B = 16384
_NC, _NS = 2, 16
_NW = _NC * _NS
_BPW = _B // _NW
_G = 16
_NGRP = _BPW // _G


def _sc_lookup(index, table3):
    mesh = plsc.VectorSubcoreMesh(core_axis_name="c", subcore_axis_name="s")

    @functools.partial(
        pl.kernel,
        mesh=mesh,
        out_type=jax.ShapeDtypeStruct((_B, _N_COL), jnp.float32),
        compiler_params=pltpu.CompilerParams(
            needs_layout_passes=False, use_tc_tiling_on_sc=True
        ),
        scratch_types=[
            pltpu.VMEM((_BPW,), jnp.int32),
            pltpu.VMEM((2, _G, 2, 8, 128), jnp.float32),
            pltpu.VMEM((2, _G, _N_COL), jnp.float32),
            pltpu.SemaphoreType.DMA,
            pltpu.SemaphoreType.DMA,
            pltpu.SemaphoreType.DMA,
            pltpu.SemaphoreType.DMA,
        ],
    )
    def body(
        idx_hbm, table_hbm, out_hbm, idx_v, tiles_v, big_v, sem0, sem1, semo0, semo1
    ):
        wid = lax.axis_index("s") * _NC + lax.axis_index("c")
        base = wid * _BPW

        pltpu.sync_copy(idx_hbm.at[pl.ds(base, _BPW)], idx_v)

        lanes = lax.iota(jnp.int32, 16)
        plane = lanes >> 3
        sub = lanes & 7
        zeros = jnp.zeros((_D,), jnp.float32)

        def fire(g, slot, sem):
            idx16 = idx_v[pl.ds(g * _G, _G)]
            for k in range(_G):
                r = idx16[k]
                r_al = pl.multiple_of((r >> 7) << 7, 128)
                pltpu.async_copy(
                    table_hbm.at[0, :, pl.ds(r_al, 128)],
                    tiles_v.at[slot, k, 0],
                    sem,
                )
                pltpu.async_copy(
                    table_hbm.at[1, :, pl.ds(r_al, 128)],
                    tiles_v.at[slot, k, 1],
                    sem,
                )

        def drain(slot, sem):
            for k in range(_G):
                pltpu.make_async_copy(
                    table_hbm.at[:, :, pl.ds(0, 128)], tiles_v.at[slot, k], sem
                ).wait()

        def drain_out(sem):
            pltpu.make_async_copy(
                out_hbm.at[pl.ds(base, _G)], big_v.at[0], sem
            ).wait()

        def extract(g, slot, sem):
            i0 = g * _G
            idx16 = idx_v[pl.ds(i0, _G)]
            off16 = idx16 & 127
            for k in range(_G):
                roff = jnp.full((16,), off16[k], jnp.int32)
                vals = plsc.load_gather(
                    tiles_v.at[slot],
                    [jnp.full((16,), k, jnp.int32), plane, sub, roff],
                )
                big_v[slot, k, pl.ds(0, _D)] = vals
                big_v[slot, k, pl.ds(_D, _D)] = zeros
            pltpu.async_copy(
                big_v.at[slot], out_hbm.at[pl.ds(base + i0, _G)], sem
            )

        fire(0, 0, sem0)

        def grp_body(g, _):
            slot = lax.rem(g, 2)

            @pl.when((g + 1 < _NGRP) & (slot == 0))
            def _fire_next1():
                fire(g + 1, 1, sem1)

            @pl.when((g + 1 < _NGRP) & (slot == 1))
            def _fire_next0():
                fire(g + 1, 0, sem0)

            @pl.when((g >= 2) & (slot == 0))
            def _drain_out0():
                drain_out(semo0)

            @pl.when((g >= 2) & (slot == 1))
            def _drain_out1():
                drain_out(semo1)

            @pl.when(slot == 0)
            def _work0():
                drain(0, sem0)
                extract(g, 0, semo0)

            @pl.when(slot == 1)
            def _work1():
                drain(1, sem1)
                extract(g, 1, semo1)

            return _

        lax.fori_loop(0, _NGRP, grp_body, None)

        drain_out(semo0)
        drain_out(semo1)

    return body(index, table3)


def kernel(index, table):
    table3 = table.T.reshape(2, 8, _VOCAB)
    return _sc_lookup(index.astype(jnp.int32), table3)
